# baseline (speedup 1.0000x reference)
import functools

import jax
import jax.numpy as jnp
from jax import lax
from jax.experimental import pallas as pl
from jax.experimental.pallas import tpu as pltpu
from jax.experimental.pallas import tpu_sc as plsc

N = 10000
E = 320000
D = 128

NC = 2
NS = 16
NW = NC * NS
L = 16

EPT = E // NW
REAL_B = 125
AGG_B = 128
AGG_NB = EPT // REAL_B
EDG_B = 125
EDG_NB = EPT // EDG_B
N_PAD = 10240
STRIPE = N_PAD // NS
DEG_PAD = 10240
DEG_STRIPE = DEG_PAD // NS

_f32 = jnp.float32


def _dotT(a, w):
    return lax.dot_general(a, w, (((1,), (1,)), ((), ())),
                           preferred_element_type=_f32)


_R = 1000
_GRID = N // _R


def _row_spec():
    return pl.BlockSpec((_R, D), lambda i: (i, 0))


def _w_spec():
    return pl.BlockSpec((D, D), lambda i: (0, 0))


def _b_spec():
    return pl.BlockSpec((1, D), lambda i: (0, 0))


def _col_spec():
    return pl.BlockSpec((_R, 1), lambda i: (i, 0))


def _stage_a_body(x_ref, wl_ref, wr_ref, bl_ref, a_ref, r_ref):
    x = x_ref[...]
    a_ref[...] = _dotT(x, wl_ref[...])
    r_ref[...] = _dotT(x, wr_ref[...]) + bl_ref[...]


def _stage_a(x, wl, wr, bl):
    return pl.pallas_call(
        _stage_a_body,
        grid=(_GRID,),
        in_specs=[_row_spec(), _w_spec(), _w_spec(), _b_spec()],
        out_specs=[_row_spec(), _row_spec()],
        out_shape=[jax.ShapeDtypeStruct((N, D), _f32)] * 2,
    )(x, wl, wr, bl.reshape(1, D))


def _part_spec(part):
    return pl.BlockSpec((1, _R, D), lambda i, part=part: (part, i, 0))


def _stage_b_body(sa_ref, sb_ref, inv_ref, r_ref, wl_ref, wr_ref, bl_ref,
                  a_ref, r2_ref):
    h = jnp.maximum((sa_ref[0] + sb_ref[0]) * inv_ref[...] + r_ref[...],
                    0.0)
    a_ref[...] = _dotT(h, wl_ref[...])
    r2_ref[...] = _dotT(h, wr_ref[...]) + bl_ref[...]


def _stage_b(sp, invd, r, wl, wr, bl):
    return pl.pallas_call(
        _stage_b_body,
        grid=(_GRID,),
        in_specs=[_part_spec(0), _part_spec(1), _col_spec(), _row_spec(),
                  _w_spec(), _w_spec(), _b_spec()],
        out_specs=[_row_spec(), _row_spec()],
        out_shape=[jax.ShapeDtypeStruct((N, D), _f32)] * 2,
    )(sp, sp, invd, r, wl, wr, bl.reshape(1, D))


def _stage_c_body(sa_ref, sb_ref, inv_ref, r_ref, w1_ref, b1_ref, w2_ref,
                  b2_ref, wa_ref, wb_ref, eb1_ref, p_ref, q_ref):
    h2 = jnp.maximum((sa_ref[0] + sb_ref[0]) * inv_ref[...] + r_ref[...],
                     0.0)
    t = jnp.maximum(_dotT(h2, w1_ref[...]) + b1_ref[...], 0.0)
    h = _dotT(t, w2_ref[...]) + b2_ref[...]
    p_ref[...] = _dotT(h, wa_ref[...]) + eb1_ref[...]
    q_ref[...] = _dotT(h, wb_ref[...])


def _stage_c(sp, invd, r, pm_w1, pm_b1, pm_w2, pm_b2, w1a, w1b, ep_b1):
    return pl.pallas_call(
        _stage_c_body,
        grid=(_GRID,),
        in_specs=[_part_spec(0), _part_spec(1), _col_spec(), _row_spec(),
                  _w_spec(), _b_spec(), _w_spec(), _b_spec(),
                  _w_spec(), _w_spec(), _b_spec()],
        out_specs=[_row_spec(), _row_spec()],
        out_shape=[jax.ShapeDtypeStruct((N, D), _f32)] * 2,
    )(sp, sp, invd, r, pm_w1, pm_b1.reshape(1, D), pm_w2,
      pm_b2.reshape(1, D), w1a, w1b, ep_b1.reshape(1, D))


_SC_MESH = dict(core_axis_name="c", subcore_axis_name="s")


def _agg_deg_kernel(a_hbm, src_hbm, dst_hbm, znd_hbm, zdeg_hbm,
                    out_hbm, deg_hbm,
                    si0, si1, di0, di1, buf0, buf1, ones, table, degtab,
                    gsem, ssem):
    c = lax.axis_index("c")
    s = lax.axis_index("s")
    wid = s * NC + c
    pltpu.sync_copy(znd_hbm.at[pl.ds(s * STRIPE, STRIPE)],
                    table.at[pl.ds(s * STRIPE, STRIPE)])
    pltpu.sync_copy(zdeg_hbm.at[pl.ds(s * DEG_STRIPE, DEG_STRIPE)],
                    degtab.at[pl.ds(s * DEG_STRIPE, DEG_STRIPE)])
    one = jnp.full((L,), 1.0, _f32)
    for off in range(0, AGG_B - L + 1, L):
        ones[pl.ds(off, L)] = one
    pltpu.sync_copy(src_hbm.at[wid, 0], si0)
    pltpu.sync_copy(dst_hbm.at[wid, 0], di0)
    pltpu.sync_copy(src_hbm.at[wid, 1], si1)
    pltpu.sync_copy(dst_hbm.at[wid, 1], di1)
    plsc.subcore_barrier()
    pltpu.async_copy(a_hbm.at[si0], buf0, gsem)

    def body(i, carry):
        j = 2 * i
        pltpu.async_copy(a_hbm.at[si1], buf1, ssem)
        pltpu.make_async_copy(a_hbm.at[si0], buf0, gsem).wait()
        pltpu.sync_copy(buf0, table.at[di0], add=True)
        pltpu.sync_copy(ones, degtab.at[di0], add=True)

        @pl.when(j + 2 < AGG_NB)
        def _():
            pltpu.sync_copy(src_hbm.at[wid, j + 2], si0)
            pltpu.sync_copy(dst_hbm.at[wid, j + 2], di0)
            pltpu.async_copy(a_hbm.at[si0], buf0, gsem)

        pltpu.make_async_copy(a_hbm.at[si1], buf1, ssem).wait()
        pltpu.sync_copy(buf1, table.at[di1], add=True)
        pltpu.sync_copy(ones, degtab.at[di1], add=True)

        @pl.when(j + 3 < AGG_NB)
        def _():
            pltpu.sync_copy(src_hbm.at[wid, j + 3], si1)
            pltpu.sync_copy(dst_hbm.at[wid, j + 3], di1)

        return carry

    lax.fori_loop(0, AGG_NB // 2, body, 0)
    plsc.subcore_barrier()
    pltpu.sync_copy(table.at[pl.ds(s * STRIPE, STRIPE)],
                    out_hbm.at[c, pl.ds(s * STRIPE, STRIPE)])
    pltpu.sync_copy(degtab.at[pl.ds(s * DEG_STRIPE, DEG_STRIPE)],
                    deg_hbm.at[c, pl.ds(s * DEG_STRIPE, DEG_STRIPE)])


def _agg_kernel(a_hbm, src_hbm, dst_hbm, znd_hbm,
                out_hbm,
                si0, si1, di0, di1, buf0, buf1, table,
                gsem, ssem):
    c = lax.axis_index("c")
    s = lax.axis_index("s")
    wid = s * NC + c
    pltpu.sync_copy(znd_hbm.at[pl.ds(s * STRIPE, STRIPE)],
                    table.at[pl.ds(s * STRIPE, STRIPE)])
    pltpu.sync_copy(src_hbm.at[wid, 0], si0)
    pltpu.sync_copy(dst_hbm.at[wid, 0], di0)
    pltpu.sync_copy(src_hbm.at[wid, 1], si1)
    pltpu.sync_copy(dst_hbm.at[wid, 1], di1)
    plsc.subcore_barrier()
    pltpu.async_copy(a_hbm.at[si0], buf0, gsem)

    def body(i, carry):
        j = 2 * i
        pltpu.async_copy(a_hbm.at[si1], buf1, ssem)
        pltpu.make_async_copy(a_hbm.at[si0], buf0, gsem).wait()
        pltpu.sync_copy(buf0, table.at[di0], add=True)

        @pl.when(j + 2 < AGG_NB)
        def _():
            pltpu.sync_copy(src_hbm.at[wid, j + 2], si0)
            pltpu.sync_copy(dst_hbm.at[wid, j + 2], di0)
            pltpu.async_copy(a_hbm.at[si0], buf0, gsem)

        pltpu.make_async_copy(a_hbm.at[si1], buf1, ssem).wait()
        pltpu.sync_copy(buf1, table.at[di1], add=True)

        @pl.when(j + 3 < AGG_NB)
        def _():
            pltpu.sync_copy(src_hbm.at[wid, j + 3], si1)
            pltpu.sync_copy(dst_hbm.at[wid, j + 3], di1)

        return carry

    lax.fori_loop(0, AGG_NB // 2, body, 0)
    plsc.subcore_barrier()
    pltpu.sync_copy(table.at[pl.ds(s * STRIPE, STRIPE)],
                    out_hbm.at[c, pl.ds(s * STRIPE, STRIPE)])


def _segment_mean_parts(a, src3, dst3, znd, zdeg, with_deg):
    if with_deg:
        out_type = [jax.ShapeDtypeStruct((NC, N_PAD, D), _f32),
                    jax.ShapeDtypeStruct((NC, DEG_PAD), _f32)]
        scratch = ([pltpu.VMEM((AGG_B,), jnp.int32)] * 4
                   + [pltpu.VMEM((AGG_B, D), _f32)] * 2
                   + [pltpu.VMEM((AGG_B,), _f32),
                      pltpu.VMEM_SHARED((N_PAD, D), _f32),
                      pltpu.VMEM_SHARED((DEG_PAD,), _f32),
                      pltpu.SemaphoreType.DMA, pltpu.SemaphoreType.DMA])
        fn = pl.kernel(_agg_deg_kernel, out_type=out_type,
                       mesh=plsc.VectorSubcoreMesh(**_SC_MESH),
                       scratch_types=scratch)
        return fn(a, src3, dst3, znd, zdeg)
    out_type = [jax.ShapeDtypeStruct((NC, N_PAD, D), _f32)]
    scratch = ([pltpu.VMEM((AGG_B,), jnp.int32)] * 4
               + [pltpu.VMEM((AGG_B, D), _f32)] * 2
               + [pltpu.VMEM_SHARED((N_PAD, D), _f32),
                  pltpu.SemaphoreType.DMA, pltpu.SemaphoreType.DMA])
    fn = pl.kernel(_agg_kernel, out_type=out_type,
                   mesh=plsc.VectorSubcoreMesh(**_SC_MESH),
                   scratch_types=scratch)
    return fn(a, src3, dst3, znd)


def _edge_kernel(p_hbm, q_hbm, pi_hbm, qi_hbm, w2_hbm,
                 y_hbm,
                 pidx, qidx, bufp0, bufq0, bufp1, bufq1, scr0, scr1, w2v,
                 semA, semB, semY):
    c = lax.axis_index("c")
    s = lax.axis_index("s")
    wid = s * NC + c
    pltpu.sync_copy(pi_hbm.at[wid], pidx)
    pltpu.sync_copy(qi_hbm.at[wid], qidx)
    pltpu.sync_copy(w2_hbm, w2v)
    zero = jnp.zeros((L,), _f32)

    def fire(j, bp, bq, sem):
        pltpu.async_copy(p_hbm.at[pidx.at[j]], bp, sem)
        pltpu.async_copy(q_hbm.at[qidx.at[j]], bq, sem)

    def drain(bp, bq, sem):
        pltpu.make_async_copy(p_hbm.at[pidx.at[0]], bp, sem).wait()
        pltpu.make_async_copy(q_hbm.at[qidx.at[0]], bq, sem).wait()

    def compute(j, bp, bq, scr):
        def edge(e, cc):
            acc = zero
            for ch in range(D // L):
                sl = pl.ds(ch * L, L)
                t = jnp.maximum(bp[e, sl] + bq[e, sl], 0.0)
                acc = acc + t * w2v[sl]
            scr[e] = acc
            return cc

        lax.fori_loop(0, EDG_B, edge, 0)
        pltpu.async_copy(scr, y_hbm.at[wid, j], semY)

    fire(0, bufp0, bufq0, semA)

    def body(i, carry):
        j = 2 * i
        fire(j + 1, bufp1, bufq1, semB)
        drain(bufp0, bufq0, semA)
        compute(j, bufp0, bufq0, scr0)

        @pl.when(j + 2 < EDG_NB)
        def _():
            fire(j + 2, bufp0, bufq0, semA)

        drain(bufp1, bufq1, semB)
        compute(j + 1, bufp1, bufq1, scr1)
        pltpu.make_async_copy(scr0, y_hbm.at[wid, 0], semY).wait()
        pltpu.make_async_copy(scr1, y_hbm.at[wid, 0], semY).wait()
        return carry

    lax.fori_loop(0, EDG_NB // 2, body, 0)


def _edge_predict(p, q, pi3, qi3, w2):
    scratch = ([pltpu.VMEM((EDG_NB, EDG_B), jnp.int32)] * 2
               + [pltpu.VMEM((EDG_B, D), _f32)] * 4
               + [pltpu.VMEM((EDG_B, L), _f32)] * 2
               + [pltpu.VMEM((D,), _f32),
                  pltpu.SemaphoreType.DMA, pltpu.SemaphoreType.DMA,
                  pltpu.SemaphoreType.DMA])
    fn = pl.kernel(_edge_kernel,
                   out_type=jax.ShapeDtypeStruct((NW, EDG_NB, EDG_B, L), _f32),
                   mesh=plsc.VectorSubcoreMesh(**_SC_MESH),
                   scratch_types=scratch)
    return fn(p, q, pi3, qi3, w2)


_ZR = 4000
_ZROWS = E * L // D


def _fold_body(z_ref, s_ref, b_ref, y_ref):
    y_ref[...] = lax.dot_general(z_ref[...], s_ref[...],
                                 (((1,), (0,)), ((), ())),
                                 preferred_element_type=_f32) + b_ref[...]


def _lane_fold(z, sel, b2):
    return pl.pallas_call(
        _fold_body,
        grid=(_ZROWS // _ZR,),
        in_specs=[pl.BlockSpec((_ZR, D), lambda i: (i, 0)),
                  pl.BlockSpec((D, 8), lambda i: (0, 0)),
                  pl.BlockSpec((1, 8), lambda i: (0, 0))],
        out_specs=pl.BlockSpec((_ZR, 8), lambda i: (i, 0)),
        out_shape=jax.ShapeDtypeStruct((_ZROWS, 8), _f32),
    )(z, sel, b2)


def kernel(x, edge_attr, edge_index, predict_edge_index,
           c1_Wl, c1_bl, c1_Wr, c2_Wl, c2_bl, c2_Wr,
           pm_W1, pm_b1, pm_W2, pm_b2,
           ep_W1, ep_b1, ep_W2, ep_b2):
    del edge_attr
    pad = ((0, 0), (0, 0), (0, AGG_B - REAL_B))
    src3 = jnp.pad(edge_index[0].reshape(NW, AGG_NB, REAL_B), pad)
    dst3 = jnp.pad(edge_index[1].reshape(NW, AGG_NB, REAL_B), pad,
                   constant_values=N_PAD - 1)
    pi3 = predict_edge_index[0].reshape(NW, EDG_NB, EDG_B)
    qi3 = predict_edge_index[1].reshape(NW, EDG_NB, EDG_B)
    znd = jnp.zeros((N_PAD, D), _f32)
    zdeg = jnp.zeros((DEG_PAD,), _f32)

    a1, r1 = _stage_a(x, c1_Wl, c1_Wr, c1_bl)
    s1p, degp = _segment_mean_parts(a1, src3, dst3, znd, zdeg, True)
    deg = (degp[0] + degp[1])[:N]
    invd = (1.0 / jnp.clip(deg, 1.0, None)).reshape(N, 1)

    a2, r2 = _stage_b(s1p, invd, r1, c2_Wl, c2_Wr, c2_bl)
    s2p = _segment_mean_parts(a2, src3, dst3, znd, None, False)[0]

    w1a = ep_W1[:, :D]
    w1b = ep_W1[:, D:]
    p, q = _stage_c(s2p, invd, r2, pm_W1, pm_b1, pm_W2, pm_b2,
                    w1a, w1b, ep_b1)

    y16 = _edge_predict(p, q, pi3, qi3, ep_W2.reshape(D))
    sel = (jnp.arange(D)[:, None] // L == jnp.arange(8)[None, :]).astype(_f32)
    b2 = jnp.broadcast_to(ep_b2.reshape(1, 1), (1, 8))
    y = _lane_fold(y16.reshape(_ZROWS, D), sel, b2)
    return y.reshape(E, 1)


# speedup vs baseline: 1.8867x; 1.8867x over previous
import functools

import jax
import jax.numpy as jnp
from jax import lax
from jax.experimental import pallas as pl
from jax.experimental.pallas import tpu as pltpu
from jax.experimental.pallas import tpu_sc as plsc

N = 10000
E = 320000
D = 128

NC = 2
NS = 16
NW = NC * NS
L = 16

EPT = E // NW
AGG_B = 125
AGG_NB = EPT // AGG_B
DPAD_B = 128
EDG_B = 125
EDG_NB = EPT // EDG_B
N_PAD = 10240
STRIPE = N_PAD // NS
DEG_PAD = 10240
DEG_STRIPE = DEG_PAD // NS

_f32 = jnp.float32


def _dotT(a, w):
    return lax.dot_general(a, w, (((1,), (1,)), ((), ())),
                           preferred_element_type=_f32)


_R = 1000
_GRID = N // _R


def _row_spec():
    return pl.BlockSpec((_R, D), lambda i: (i, 0))


def _w_spec():
    return pl.BlockSpec((D, D), lambda i: (0, 0))


def _b_spec():
    return pl.BlockSpec((1, D), lambda i: (0, 0))


def _col_spec():
    return pl.BlockSpec((_R, 1), lambda i: (i, 0))


def _stage_a_body(x_ref, wl_ref, wr_ref, bl_ref, a_ref, r_ref):
    x = x_ref[...]
    a_ref[...] = _dotT(x, wl_ref[...])
    r_ref[...] = _dotT(x, wr_ref[...]) + bl_ref[...]


def _stage_a(x, wl, wr, bl):
    return pl.pallas_call(
        _stage_a_body,
        grid=(_GRID,),
        in_specs=[_row_spec(), _w_spec(), _w_spec(), _b_spec()],
        out_specs=[_row_spec(), _row_spec()],
        out_shape=[jax.ShapeDtypeStruct((N, D), _f32)] * 2,
    )(x, wl, wr, bl.reshape(1, D))


def _part_spec(part):
    return pl.BlockSpec((1, _R, D), lambda i, part=part: (part, i, 0))


def _stage_b_body(sa_ref, sb_ref, inv_ref, r_ref, wl_ref, wr_ref, bl_ref,
                  a_ref, r2_ref):
    h = jnp.maximum((sa_ref[0] + sb_ref[0]) * inv_ref[...] + r_ref[...],
                    0.0)
    a_ref[...] = _dotT(h, wl_ref[...])
    r2_ref[...] = _dotT(h, wr_ref[...]) + bl_ref[...]


def _stage_b(sp, invd, r, wl, wr, bl):
    return pl.pallas_call(
        _stage_b_body,
        grid=(_GRID,),
        in_specs=[_part_spec(0), _part_spec(1), _col_spec(), _row_spec(),
                  _w_spec(), _w_spec(), _b_spec()],
        out_specs=[_row_spec(), _row_spec()],
        out_shape=[jax.ShapeDtypeStruct((N, D), _f32)] * 2,
    )(sp, sp, invd, r, wl, wr, bl.reshape(1, D))


def _stage_c_body(sa_ref, sb_ref, inv_ref, r_ref, w1_ref, b1_ref, w2_ref,
                  b2_ref, wa_ref, wb_ref, eb1_ref, p_ref, q_ref):
    h2 = jnp.maximum((sa_ref[0] + sb_ref[0]) * inv_ref[...] + r_ref[...],
                     0.0)
    t = jnp.maximum(_dotT(h2, w1_ref[...]) + b1_ref[...], 0.0)
    h = _dotT(t, w2_ref[...]) + b2_ref[...]
    p_ref[...] = _dotT(h, wa_ref[...]) + eb1_ref[...]
    q_ref[...] = _dotT(h, wb_ref[...])


def _stage_c(sp, invd, r, pm_w1, pm_b1, pm_w2, pm_b2, w1a, w1b, ep_b1):
    return pl.pallas_call(
        _stage_c_body,
        grid=(_GRID,),
        in_specs=[_part_spec(0), _part_spec(1), _col_spec(), _row_spec(),
                  _w_spec(), _b_spec(), _w_spec(), _b_spec(),
                  _w_spec(), _w_spec(), _b_spec()],
        out_specs=[_row_spec(), _row_spec()],
        out_shape=[jax.ShapeDtypeStruct((N, D), _f32)] * 2,
    )(sp, sp, invd, r, pm_w1, pm_b1.reshape(1, D), pm_w2,
      pm_b2.reshape(1, D), w1a, w1b, ep_b1.reshape(1, D))


_SC_MESH = dict(core_axis_name="c", subcore_axis_name="s")


def _agg_deg_kernel(a_hbm, src_hbm, dst_hbm, znd_hbm, zdeg_hbm,
                    out_hbm, deg_hbm,
                    sidx, db0, db1, buf0, buf1, ones, table, degtab,
                    gsem, ssem, dsem0, dsem1):
    c = lax.axis_index("c")
    s = lax.axis_index("s")
    wid = s * NC + c
    pltpu.sync_copy(src_hbm.at[wid], sidx)
    pltpu.sync_copy(dst_hbm.at[wid, 0], db0)
    pltpu.sync_copy(dst_hbm.at[wid, 1], db1)
    pltpu.sync_copy(znd_hbm.at[pl.ds(s * STRIPE, STRIPE)],
                    table.at[pl.ds(s * STRIPE, STRIPE)])
    pltpu.sync_copy(zdeg_hbm.at[pl.ds(s * DEG_STRIPE, DEG_STRIPE)],
                    degtab.at[pl.ds(s * DEG_STRIPE, DEG_STRIPE)])
    one = jnp.full((L,), 1.0, _f32)
    for off in range(0, AGG_B - L + 1, L):
        ones[pl.ds(off, L)] = one
    ones[pl.ds(AGG_B - L, L)] = one
    plsc.subcore_barrier()
    pltpu.async_copy(a_hbm.at[sidx.at[0]], buf0, gsem)

    def body(i, carry):
        j = 2 * i
        pltpu.async_copy(a_hbm.at[sidx.at[j + 1]], buf1, ssem)
        pltpu.make_async_copy(a_hbm.at[sidx.at[0]], buf0, gsem).wait()

        @pl.when(j > 0)
        def _():
            pltpu.make_async_copy(dst_hbm.at[wid, 0], db0, dsem0).wait()

        d0 = db0.at[pl.ds(0, AGG_B)]
        pltpu.sync_copy(buf0, table.at[d0], add=True)
        pltpu.sync_copy(ones, degtab.at[d0], add=True)

        @pl.when(j + 2 < AGG_NB)
        def _():
            pltpu.async_copy(dst_hbm.at[wid, j + 2], db0, dsem0)
            pltpu.async_copy(a_hbm.at[sidx.at[j + 2]], buf0, gsem)

        pltpu.make_async_copy(a_hbm.at[sidx.at[0]], buf1, ssem).wait()

        @pl.when(j > 0)
        def _():
            pltpu.make_async_copy(dst_hbm.at[wid, 0], db1, dsem1).wait()

        d1 = db1.at[pl.ds(0, AGG_B)]
        pltpu.sync_copy(buf1, table.at[d1], add=True)
        pltpu.sync_copy(ones, degtab.at[d1], add=True)

        @pl.when(j + 3 < AGG_NB)
        def _():
            pltpu.async_copy(dst_hbm.at[wid, j + 3], db1, dsem1)

        return carry

    lax.fori_loop(0, AGG_NB // 2, body, 0)
    plsc.subcore_barrier()
    pltpu.sync_copy(table.at[pl.ds(s * STRIPE, STRIPE)],
                    out_hbm.at[c, pl.ds(s * STRIPE, STRIPE)])
    pltpu.sync_copy(degtab.at[pl.ds(s * DEG_STRIPE, DEG_STRIPE)],
                    deg_hbm.at[c, pl.ds(s * DEG_STRIPE, DEG_STRIPE)])


def _agg_kernel(a_hbm, src_hbm, dst_hbm, znd_hbm,
                out_hbm,
                sidx, db0, db1, buf0, buf1, table,
                gsem, ssem, dsem0, dsem1):
    c = lax.axis_index("c")
    s = lax.axis_index("s")
    wid = s * NC + c
    pltpu.sync_copy(src_hbm.at[wid], sidx)
    pltpu.sync_copy(dst_hbm.at[wid, 0], db0)
    pltpu.sync_copy(dst_hbm.at[wid, 1], db1)
    pltpu.sync_copy(znd_hbm.at[pl.ds(s * STRIPE, STRIPE)],
                    table.at[pl.ds(s * STRIPE, STRIPE)])
    plsc.subcore_barrier()
    pltpu.async_copy(a_hbm.at[sidx.at[0]], buf0, gsem)

    def body(i, carry):
        j = 2 * i
        pltpu.async_copy(a_hbm.at[sidx.at[j + 1]], buf1, ssem)
        pltpu.make_async_copy(a_hbm.at[sidx.at[0]], buf0, gsem).wait()

        @pl.when(j > 0)
        def _():
            pltpu.make_async_copy(dst_hbm.at[wid, 0], db0, dsem0).wait()

        pltpu.sync_copy(buf0, table.at[db0.at[pl.ds(0, AGG_B)]], add=True)

        @pl.when(j + 2 < AGG_NB)
        def _():
            pltpu.async_copy(dst_hbm.at[wid, j + 2], db0, dsem0)
            pltpu.async_copy(a_hbm.at[sidx.at[j + 2]], buf0, gsem)

        pltpu.make_async_copy(a_hbm.at[sidx.at[0]], buf1, ssem).wait()

        @pl.when(j > 0)
        def _():
            pltpu.make_async_copy(dst_hbm.at[wid, 0], db1, dsem1).wait()

        pltpu.sync_copy(buf1, table.at[db1.at[pl.ds(0, AGG_B)]], add=True)

        @pl.when(j + 3 < AGG_NB)
        def _():
            pltpu.async_copy(dst_hbm.at[wid, j + 3], db1, dsem1)

        return carry

    lax.fori_loop(0, AGG_NB // 2, body, 0)
    plsc.subcore_barrier()
    pltpu.sync_copy(table.at[pl.ds(s * STRIPE, STRIPE)],
                    out_hbm.at[c, pl.ds(s * STRIPE, STRIPE)])


def _segment_mean_parts(a, src3, dst3, znd, zdeg, with_deg):
    if with_deg:
        out_type = [jax.ShapeDtypeStruct((NC, N_PAD, D), _f32),
                    jax.ShapeDtypeStruct((NC, DEG_PAD), _f32)]
        scratch = ([pltpu.VMEM((AGG_NB, AGG_B), jnp.int32)]
                   + [pltpu.VMEM((DPAD_B,), jnp.int32)] * 2
                   + [pltpu.VMEM((AGG_B, D), _f32)] * 2
                   + [pltpu.VMEM((AGG_B,), _f32),
                      pltpu.VMEM_SHARED((N_PAD, D), _f32),
                      pltpu.VMEM_SHARED((DEG_PAD,), _f32)]
                   + [pltpu.SemaphoreType.DMA] * 4)
        fn = pl.kernel(_agg_deg_kernel, out_type=out_type,
                       mesh=plsc.VectorSubcoreMesh(**_SC_MESH),
                       scratch_types=scratch)
        return fn(a, src3, dst3, znd, zdeg)
    out_type = [jax.ShapeDtypeStruct((NC, N_PAD, D), _f32)]
    scratch = ([pltpu.VMEM((AGG_NB, AGG_B), jnp.int32)]
               + [pltpu.VMEM((DPAD_B,), jnp.int32)] * 2
               + [pltpu.VMEM((AGG_B, D), _f32)] * 2
               + [pltpu.VMEM_SHARED((N_PAD, D), _f32)]
               + [pltpu.SemaphoreType.DMA] * 4)
    fn = pl.kernel(_agg_kernel, out_type=out_type,
                   mesh=plsc.VectorSubcoreMesh(**_SC_MESH),
                   scratch_types=scratch)
    return fn(a, src3, dst3, znd)


def _edge_kernel(p_hbm, q_hbm, pi_hbm, qi_hbm, w2_hbm,
                 y_hbm,
                 pidx, qidx, bufp0, bufq0, bufp1, bufq1, scr0, scr1, w2v,
                 semA, semB, semY):
    c = lax.axis_index("c")
    s = lax.axis_index("s")
    wid = s * NC + c
    pltpu.sync_copy(pi_hbm.at[wid], pidx)
    pltpu.sync_copy(qi_hbm.at[wid], qidx)
    pltpu.sync_copy(w2_hbm, w2v)
    zero = jnp.zeros((L,), _f32)

    def fire(j, bp, bq, sem):
        pltpu.async_copy(p_hbm.at[pidx.at[j]], bp, sem)
        pltpu.async_copy(q_hbm.at[qidx.at[j]], bq, sem)

    def drain(bp, bq, sem):
        pltpu.make_async_copy(p_hbm.at[pidx.at[0]], bp, sem).wait()
        pltpu.make_async_copy(q_hbm.at[qidx.at[0]], bq, sem).wait()

    def compute(j, bp, bq, scr):
        def edge(e, cc):
            acc = zero
            for ch in range(D // L):
                sl = pl.ds(ch * L, L)
                t = jnp.maximum(bp[e, sl] + bq[e, sl], 0.0)
                acc = acc + t * w2v[sl]
            scr[e] = acc
            return cc

        lax.fori_loop(0, EDG_B, edge, 0)
        pltpu.async_copy(scr, y_hbm.at[wid, j], semY)

    fire(0, bufp0, bufq0, semA)

    def body(i, carry):
        j = 2 * i
        fire(j + 1, bufp1, bufq1, semB)
        drain(bufp0, bufq0, semA)
        compute(j, bufp0, bufq0, scr0)

        @pl.when(j + 2 < EDG_NB)
        def _():
            fire(j + 2, bufp0, bufq0, semA)

        drain(bufp1, bufq1, semB)
        compute(j + 1, bufp1, bufq1, scr1)
        pltpu.make_async_copy(scr0, y_hbm.at[wid, 0], semY).wait()
        pltpu.make_async_copy(scr1, y_hbm.at[wid, 0], semY).wait()
        return carry

    lax.fori_loop(0, EDG_NB // 2, body, 0)


def _edge_predict(p, q, pi3, qi3, w2):
    scratch = ([pltpu.VMEM((EDG_NB, EDG_B), jnp.int32)] * 2
               + [pltpu.VMEM((EDG_B, D), _f32)] * 4
               + [pltpu.VMEM((EDG_B, L), _f32)] * 2
               + [pltpu.VMEM((D,), _f32),
                  pltpu.SemaphoreType.DMA, pltpu.SemaphoreType.DMA,
                  pltpu.SemaphoreType.DMA])
    fn = pl.kernel(_edge_kernel,
                   out_type=jax.ShapeDtypeStruct((NW, EDG_NB, EDG_B, L), _f32),
                   mesh=plsc.VectorSubcoreMesh(**_SC_MESH),
                   scratch_types=scratch)
    return fn(p, q, pi3, qi3, w2)


_ZR = 4000
_ZROWS = E * L // D


def _fold_body(z_ref, s_ref, b_ref, y_ref):
    y_ref[...] = lax.dot_general(z_ref[...], s_ref[...],
                                 (((1,), (0,)), ((), ())),
                                 preferred_element_type=_f32) + b_ref[...]


def _lane_fold(z, sel, b2):
    return pl.pallas_call(
        _fold_body,
        grid=(_ZROWS // _ZR,),
        in_specs=[pl.BlockSpec((_ZR, D), lambda i: (i, 0)),
                  pl.BlockSpec((D, 8), lambda i: (0, 0)),
                  pl.BlockSpec((1, 8), lambda i: (0, 0))],
        out_specs=pl.BlockSpec((_ZR, 8), lambda i: (i, 0)),
        out_shape=jax.ShapeDtypeStruct((_ZROWS, 8), _f32),
    )(z, sel, b2)


def kernel(x, edge_attr, edge_index, predict_edge_index,
           c1_Wl, c1_bl, c1_Wr, c2_Wl, c2_bl, c2_Wr,
           pm_W1, pm_b1, pm_W2, pm_b2,
           ep_W1, ep_b1, ep_W2, ep_b2):
    del edge_attr
    src3 = edge_index[0].reshape(NW, AGG_NB, AGG_B)
    dst3 = jnp.pad(edge_index[1].reshape(NW, AGG_NB, AGG_B),
                   ((0, 0), (0, 0), (0, DPAD_B - AGG_B)))
    pi3 = predict_edge_index[0].reshape(NW, EDG_NB, EDG_B)
    qi3 = predict_edge_index[1].reshape(NW, EDG_NB, EDG_B)
    znd = jnp.zeros((N_PAD, D), _f32)
    zdeg = jnp.zeros((DEG_PAD,), _f32)

    a1, r1 = _stage_a(x, c1_Wl, c1_Wr, c1_bl)
    s1p, degp = _segment_mean_parts(a1, src3, dst3, znd, zdeg, True)
    deg = (degp[0] + degp[1])[:N]
    invd = (1.0 / jnp.clip(deg, 1.0, None)).reshape(N, 1)

    a2, r2 = _stage_b(s1p, invd, r1, c2_Wl, c2_Wr, c2_bl)
    s2p = _segment_mean_parts(a2, src3, dst3, znd, None, False)[0]

    w1a = ep_W1[:, :D]
    w1b = ep_W1[:, D:]
    p, q = _stage_c(s2p, invd, r2, pm_W1, pm_b1, pm_W2, pm_b2,
                    w1a, w1b, ep_b1)

    y16 = _edge_predict(p, q, pi3, qi3, ep_W2.reshape(D))
    sel = (jnp.arange(D)[:, None] // L == jnp.arange(8)[None, :]).astype(_f32)
    b2 = jnp.broadcast_to(ep_b2.reshape(1, 1), (1, 8))
    y = _lane_fold(y16.reshape(_ZROWS, D), sel, b2)
    return y.reshape(E, 1)


# speedup vs baseline: 1.8877x; 1.0005x over previous
import functools

import jax
import jax.numpy as jnp
from jax import lax
from jax.experimental import pallas as pl
from jax.experimental.pallas import tpu as pltpu
from jax.experimental.pallas import tpu_sc as plsc

N = 10000
E = 320000
D = 128

NC = 2
NS = 16
NW = NC * NS
L = 16

EPT = E // NW
AGG_B = 125
AGG_NB = EPT // AGG_B
DPAD_B = 128
EDG_B = 125
EDG_NB = EPT // EDG_B
N_PAD = 10240
STRIPE = N_PAD // NS
DEG_PAD = 10240
DEG_STRIPE = DEG_PAD // NS

_f32 = jnp.float32


def _dotT(a, w):
    return lax.dot_general(a, w, (((1,), (1,)), ((), ())),
                           preferred_element_type=_f32)


_R = 1000
_GRID = N // _R


def _row_spec():
    return pl.BlockSpec((_R, D), lambda i: (i, 0))


def _w_spec():
    return pl.BlockSpec((D, D), lambda i: (0, 0))


def _b_spec():
    return pl.BlockSpec((1, D), lambda i: (0, 0))


def _col_spec():
    return pl.BlockSpec((_R, 1), lambda i: (i, 0))


def _stage_a1_body(x_ref, wl_ref, a_ref):
    a_ref[...] = _dotT(x_ref[...], wl_ref[...])


def _stage_a1(x, wl):
    return pl.pallas_call(
        _stage_a1_body,
        grid=(_GRID,),
        in_specs=[_row_spec(), _w_spec()],
        out_specs=_row_spec(),
        out_shape=jax.ShapeDtypeStruct((N, D), _f32),
    )(x, wl)


def _stage_a2_body(x_ref, wr_ref, bl_ref, r_ref):
    r_ref[...] = _dotT(x_ref[...], wr_ref[...]) + bl_ref[...]


def _stage_a2(x, wr, bl):
    return pl.pallas_call(
        _stage_a2_body,
        grid=(_GRID,),
        in_specs=[_row_spec(), _w_spec(), _b_spec()],
        out_specs=_row_spec(),
        out_shape=jax.ShapeDtypeStruct((N, D), _f32),
    )(x, wr, bl.reshape(1, D))


def _part_spec(part):
    return pl.BlockSpec((1, _R, D), lambda i, part=part: (part, i, 0))


def _stage_b1_body(sa_ref, sb_ref, inv_ref, r_ref, wl_ref, a_ref):
    h = jnp.maximum((sa_ref[0] + sb_ref[0]) * inv_ref[...] + r_ref[...],
                    0.0)
    a_ref[...] = _dotT(h, wl_ref[...])


def _stage_b1(sp, invd, r, wl):
    return pl.pallas_call(
        _stage_b1_body,
        grid=(_GRID,),
        in_specs=[_part_spec(0), _part_spec(1), _col_spec(), _row_spec(),
                  _w_spec()],
        out_specs=_row_spec(),
        out_shape=jax.ShapeDtypeStruct((N, D), _f32),
    )(sp, sp, invd, r, wl)


def _stage_b2_body(sa_ref, sb_ref, inv_ref, r_ref, wr_ref, bl_ref, r2_ref):
    h = jnp.maximum((sa_ref[0] + sb_ref[0]) * inv_ref[...] + r_ref[...],
                    0.0)
    r2_ref[...] = _dotT(h, wr_ref[...]) + bl_ref[...]


def _stage_b2(sp, invd, r, wr, bl):
    return pl.pallas_call(
        _stage_b2_body,
        grid=(_GRID,),
        in_specs=[_part_spec(0), _part_spec(1), _col_spec(), _row_spec(),
                  _w_spec(), _b_spec()],
        out_specs=_row_spec(),
        out_shape=jax.ShapeDtypeStruct((N, D), _f32),
    )(sp, sp, invd, r, wr, bl.reshape(1, D))


def _stage_c_body(sa_ref, sb_ref, inv_ref, r_ref, w1_ref, b1_ref, w2_ref,
                  b2_ref, wa_ref, wb_ref, eb1_ref, p_ref, q_ref):
    h2 = jnp.maximum((sa_ref[0] + sb_ref[0]) * inv_ref[...] + r_ref[...],
                     0.0)
    t = jnp.maximum(_dotT(h2, w1_ref[...]) + b1_ref[...], 0.0)
    h = _dotT(t, w2_ref[...]) + b2_ref[...]
    p_ref[...] = _dotT(h, wa_ref[...]) + eb1_ref[...]
    q_ref[...] = _dotT(h, wb_ref[...])


def _stage_c(sp, invd, r, pm_w1, pm_b1, pm_w2, pm_b2, w1a, w1b, ep_b1):
    return pl.pallas_call(
        _stage_c_body,
        grid=(_GRID,),
        in_specs=[_part_spec(0), _part_spec(1), _col_spec(), _row_spec(),
                  _w_spec(), _b_spec(), _w_spec(), _b_spec(),
                  _w_spec(), _w_spec(), _b_spec()],
        out_specs=[_row_spec(), _row_spec()],
        out_shape=[jax.ShapeDtypeStruct((N, D), _f32)] * 2,
    )(sp, sp, invd, r, pm_w1, pm_b1.reshape(1, D), pm_w2,
      pm_b2.reshape(1, D), w1a, w1b, ep_b1.reshape(1, D))


_SC_MESH = dict(core_axis_name="c", subcore_axis_name="s")


def _agg_deg_kernel(a_hbm, src_hbm, dst_hbm, znd_hbm, zdeg_hbm,
                    out_hbm, deg_hbm,
                    sidx, db0, db1, buf0, buf1, ones, table, degtab,
                    gsem, ssem, dsem0, dsem1):
    c = lax.axis_index("c")
    s = lax.axis_index("s")
    wid = s * NC + c
    pltpu.sync_copy(src_hbm.at[wid], sidx)
    pltpu.sync_copy(dst_hbm.at[wid, 0], db0)
    pltpu.sync_copy(dst_hbm.at[wid, 1], db1)
    pltpu.sync_copy(znd_hbm.at[pl.ds(s * STRIPE, STRIPE)],
                    table.at[pl.ds(s * STRIPE, STRIPE)])
    pltpu.sync_copy(zdeg_hbm.at[pl.ds(s * DEG_STRIPE, DEG_STRIPE)],
                    degtab.at[pl.ds(s * DEG_STRIPE, DEG_STRIPE)])
    one = jnp.full((L,), 1.0, _f32)
    for off in range(0, AGG_B - L + 1, L):
        ones[pl.ds(off, L)] = one
    ones[pl.ds(AGG_B - L, L)] = one
    plsc.subcore_barrier()
    pltpu.async_copy(a_hbm.at[sidx.at[0]], buf0, gsem)

    def body(i, carry):
        j = 2 * i
        pltpu.async_copy(a_hbm.at[sidx.at[j + 1]], buf1, ssem)
        pltpu.make_async_copy(a_hbm.at[sidx.at[0]], buf0, gsem).wait()

        @pl.when(j > 0)
        def _():
            pltpu.make_async_copy(dst_hbm.at[wid, 0], db0, dsem0).wait()

        d0 = db0.at[pl.ds(0, AGG_B)]
        pltpu.sync_copy(buf0, table.at[d0], add=True)
        pltpu.sync_copy(ones, degtab.at[d0], add=True)

        @pl.when(j + 2 < AGG_NB)
        def _():
            pltpu.async_copy(dst_hbm.at[wid, j + 2], db0, dsem0)
            pltpu.async_copy(a_hbm.at[sidx.at[j + 2]], buf0, gsem)

        pltpu.make_async_copy(a_hbm.at[sidx.at[0]], buf1, ssem).wait()

        @pl.when(j > 0)
        def _():
            pltpu.make_async_copy(dst_hbm.at[wid, 0], db1, dsem1).wait()

        d1 = db1.at[pl.ds(0, AGG_B)]
        pltpu.sync_copy(buf1, table.at[d1], add=True)
        pltpu.sync_copy(ones, degtab.at[d1], add=True)

        @pl.when(j + 3 < AGG_NB)
        def _():
            pltpu.async_copy(dst_hbm.at[wid, j + 3], db1, dsem1)

        return carry

    lax.fori_loop(0, AGG_NB // 2, body, 0)
    plsc.subcore_barrier()
    pltpu.sync_copy(table.at[pl.ds(s * STRIPE, STRIPE)],
                    out_hbm.at[c, pl.ds(s * STRIPE, STRIPE)])
    pltpu.sync_copy(degtab.at[pl.ds(s * DEG_STRIPE, DEG_STRIPE)],
                    deg_hbm.at[c, pl.ds(s * DEG_STRIPE, DEG_STRIPE)])


def _agg_kernel(a_hbm, src_hbm, dst_hbm, znd_hbm,
                out_hbm,
                sidx, db0, db1, buf0, buf1, table,
                gsem, ssem, dsem0, dsem1):
    c = lax.axis_index("c")
    s = lax.axis_index("s")
    wid = s * NC + c
    pltpu.sync_copy(src_hbm.at[wid], sidx)
    pltpu.sync_copy(dst_hbm.at[wid, 0], db0)
    pltpu.sync_copy(dst_hbm.at[wid, 1], db1)
    pltpu.sync_copy(znd_hbm.at[pl.ds(s * STRIPE, STRIPE)],
                    table.at[pl.ds(s * STRIPE, STRIPE)])
    plsc.subcore_barrier()
    pltpu.async_copy(a_hbm.at[sidx.at[0]], buf0, gsem)

    def body(i, carry):
        j = 2 * i
        pltpu.async_copy(a_hbm.at[sidx.at[j + 1]], buf1, ssem)
        pltpu.make_async_copy(a_hbm.at[sidx.at[0]], buf0, gsem).wait()

        @pl.when(j > 0)
        def _():
            pltpu.make_async_copy(dst_hbm.at[wid, 0], db0, dsem0).wait()

        pltpu.sync_copy(buf0, table.at[db0.at[pl.ds(0, AGG_B)]], add=True)

        @pl.when(j + 2 < AGG_NB)
        def _():
            pltpu.async_copy(dst_hbm.at[wid, j + 2], db0, dsem0)
            pltpu.async_copy(a_hbm.at[sidx.at[j + 2]], buf0, gsem)

        pltpu.make_async_copy(a_hbm.at[sidx.at[0]], buf1, ssem).wait()

        @pl.when(j > 0)
        def _():
            pltpu.make_async_copy(dst_hbm.at[wid, 0], db1, dsem1).wait()

        pltpu.sync_copy(buf1, table.at[db1.at[pl.ds(0, AGG_B)]], add=True)

        @pl.when(j + 3 < AGG_NB)
        def _():
            pltpu.async_copy(dst_hbm.at[wid, j + 3], db1, dsem1)

        return carry

    lax.fori_loop(0, AGG_NB // 2, body, 0)
    plsc.subcore_barrier()
    pltpu.sync_copy(table.at[pl.ds(s * STRIPE, STRIPE)],
                    out_hbm.at[c, pl.ds(s * STRIPE, STRIPE)])


def _segment_mean_parts(a, src3, dst3, znd, zdeg, with_deg):
    if with_deg:
        out_type = [jax.ShapeDtypeStruct((NC, N_PAD, D), _f32),
                    jax.ShapeDtypeStruct((NC, DEG_PAD), _f32)]
        scratch = ([pltpu.VMEM((AGG_NB, AGG_B), jnp.int32)]
                   + [pltpu.VMEM((DPAD_B,), jnp.int32)] * 2
                   + [pltpu.VMEM((AGG_B, D), _f32)] * 2
                   + [pltpu.VMEM((AGG_B,), _f32),
                      pltpu.VMEM_SHARED((N_PAD, D), _f32),
                      pltpu.VMEM_SHARED((DEG_PAD,), _f32)]
                   + [pltpu.SemaphoreType.DMA] * 4)
        fn = pl.kernel(_agg_deg_kernel, out_type=out_type,
                       mesh=plsc.VectorSubcoreMesh(**_SC_MESH),
                       scratch_types=scratch)
        return fn(a, src3, dst3, znd, zdeg)
    out_type = [jax.ShapeDtypeStruct((NC, N_PAD, D), _f32)]
    scratch = ([pltpu.VMEM((AGG_NB, AGG_B), jnp.int32)]
               + [pltpu.VMEM((DPAD_B,), jnp.int32)] * 2
               + [pltpu.VMEM((AGG_B, D), _f32)] * 2
               + [pltpu.VMEM_SHARED((N_PAD, D), _f32)]
               + [pltpu.SemaphoreType.DMA] * 4)
    fn = pl.kernel(_agg_kernel, out_type=out_type,
                   mesh=plsc.VectorSubcoreMesh(**_SC_MESH),
                   scratch_types=scratch)
    return fn(a, src3, dst3, znd)


def _edge_kernel(p_hbm, q_hbm, pi_hbm, qi_hbm, w2_hbm,
                 y_hbm,
                 pidx, qidx, bufp0, bufq0, bufp1, bufq1, scr0, scr1, w2v,
                 semA, semB, semY):
    c = lax.axis_index("c")
    s = lax.axis_index("s")
    wid = s * NC + c
    pltpu.sync_copy(pi_hbm.at[wid], pidx)
    pltpu.sync_copy(qi_hbm.at[wid], qidx)
    pltpu.sync_copy(w2_hbm, w2v)
    zero = jnp.zeros((L,), _f32)

    def fire(j, bp, bq, sem):
        pltpu.async_copy(p_hbm.at[pidx.at[j]], bp, sem)
        pltpu.async_copy(q_hbm.at[qidx.at[j]], bq, sem)

    def drain(bp, bq, sem):
        pltpu.make_async_copy(p_hbm.at[pidx.at[0]], bp, sem).wait()
        pltpu.make_async_copy(q_hbm.at[qidx.at[0]], bq, sem).wait()

    def compute(j, bp, bq, scr):
        def edge(e, cc):
            acc = zero
            for ch in range(D // L):
                sl = pl.ds(ch * L, L)
                t = jnp.maximum(bp[e, sl] + bq[e, sl], 0.0)
                acc = acc + t * w2v[sl]
            scr[e] = acc
            return cc

        lax.fori_loop(0, EDG_B, edge, 0)
        pltpu.async_copy(scr, y_hbm.at[wid, j], semY)

    fire(0, bufp0, bufq0, semA)

    def body(i, carry):
        j = 2 * i
        fire(j + 1, bufp1, bufq1, semB)
        drain(bufp0, bufq0, semA)
        compute(j, bufp0, bufq0, scr0)

        @pl.when(j + 2 < EDG_NB)
        def _():
            fire(j + 2, bufp0, bufq0, semA)

        drain(bufp1, bufq1, semB)
        compute(j + 1, bufp1, bufq1, scr1)
        pltpu.make_async_copy(scr0, y_hbm.at[wid, 0], semY).wait()
        pltpu.make_async_copy(scr1, y_hbm.at[wid, 0], semY).wait()
        return carry

    lax.fori_loop(0, EDG_NB // 2, body, 0)


def _edge_predict(p, q, pi3, qi3, w2):
    scratch = ([pltpu.VMEM((EDG_NB, EDG_B), jnp.int32)] * 2
               + [pltpu.VMEM((EDG_B, D), _f32)] * 4
               + [pltpu.VMEM((EDG_B, L), _f32)] * 2
               + [pltpu.VMEM((D,), _f32),
                  pltpu.SemaphoreType.DMA, pltpu.SemaphoreType.DMA,
                  pltpu.SemaphoreType.DMA])
    fn = pl.kernel(_edge_kernel,
                   out_type=jax.ShapeDtypeStruct((NW, EDG_NB, EDG_B, L), _f32),
                   mesh=plsc.VectorSubcoreMesh(**_SC_MESH),
                   scratch_types=scratch)
    return fn(p, q, pi3, qi3, w2)


_ZR = 4000
_ZROWS = E * L // D


def _fold_body(z_ref, b_ref, y_ref):
    row = lax.broadcasted_iota(jnp.int32, (D, 8), 0) // L
    col = lax.broadcasted_iota(jnp.int32, (D, 8), 1)
    sel = (row == col).astype(_f32)
    y_ref[...] = lax.dot_general(z_ref[...], sel,
                                 (((1,), (0,)), ((), ())),
                                 preferred_element_type=_f32) + b_ref[...]


def _lane_fold(z, b2):
    return pl.pallas_call(
        _fold_body,
        grid=(_ZROWS // _ZR,),
        in_specs=[pl.BlockSpec((_ZR, D), lambda i: (i, 0)),
                  pl.BlockSpec((1, 8), lambda i: (0, 0))],
        out_specs=pl.BlockSpec((_ZR, 8), lambda i: (i, 0)),
        out_shape=jax.ShapeDtypeStruct((_ZROWS, 8), _f32),
    )(z, b2)


def kernel(x, edge_attr, edge_index, predict_edge_index,
           c1_Wl, c1_bl, c1_Wr, c2_Wl, c2_bl, c2_Wr,
           pm_W1, pm_b1, pm_W2, pm_b2,
           ep_W1, ep_b1, ep_W2, ep_b2):
    del edge_attr
    src3 = edge_index[0].reshape(NW, AGG_NB, AGG_B)
    dst3 = jnp.pad(edge_index[1].reshape(NW, AGG_NB, AGG_B),
                   ((0, 0), (0, 0), (0, DPAD_B - AGG_B)))
    pi3 = predict_edge_index[0].reshape(NW, EDG_NB, EDG_B)
    qi3 = predict_edge_index[1].reshape(NW, EDG_NB, EDG_B)
    znd = jnp.zeros((N_PAD, D), _f32)
    zdeg = jnp.zeros((DEG_PAD,), _f32)

    a1 = _stage_a1(x, c1_Wl)
    s1p, degp = _segment_mean_parts(a1, src3, dst3, znd, zdeg, True)
    r1 = _stage_a2(x, c1_Wr, c1_bl)
    deg = (degp[0] + degp[1])[:N]
    invd = (1.0 / jnp.clip(deg, 1.0, None)).reshape(N, 1)

    a2 = _stage_b1(s1p, invd, r1, c2_Wl)
    s2p = _segment_mean_parts(a2, src3, dst3, znd, None, False)[0]
    r2 = _stage_b2(s1p, invd, r1, c2_Wr, c2_bl)

    w1a = ep_W1[:, :D]
    w1b = ep_W1[:, D:]
    p, q = _stage_c(s2p, invd, r2, pm_W1, pm_b1, pm_W2, pm_b2,
                    w1a, w1b, ep_b1)

    y16 = _edge_predict(p, q, pi3, qi3, ep_W2.reshape(D))
    b2 = jnp.broadcast_to(ep_b2.reshape(1, 1), (1, 8))
    y = _lane_fold(y16.reshape(_ZROWS, D), b2)
    return y.reshape(E, 1)
